# baseline (speedup 1.0000x reference)
import functools

import jax
import jax.numpy as jnp
from jax import lax
from jax.experimental import pallas as pl
from jax.experimental.pallas import tpu as pltpu
from jax.experimental.pallas import tpu_sc as plsc

TOKENS = 16384
D_MODEL = 4096
N_EXPERTS = 64
BLK = 1024

NC, NS, L = 2, 16, 16
NW = NC * NS
TPW = TOKENS // NW
GROUPS = TPW // L


def _matmul_block(x_ref, w_ref, s_ref):
    s_ref[...] = jax.lax.dot_general(
        w_ref[...], x_ref[...], (((1,), (1,)), ((), ())),
        preferred_element_type=jnp.float32,
        precision=jax.lax.Precision.DEFAULT,
    )


def _scores_t(x, W):
    return pl.pallas_call(
        _matmul_block,
        grid=(TOKENS // BLK,),
        in_specs=[
            pl.BlockSpec((BLK, D_MODEL), lambda i: (i, 0)),
            pl.BlockSpec((N_EXPERTS, D_MODEL), lambda i: (0, 0)),
        ],
        out_specs=pl.BlockSpec((N_EXPERTS, BLK), lambda i: (0, i)),
        out_shape=jax.ShapeDtypeStruct((N_EXPERTS, TOKENS), jnp.float32),
    )(x, W)


def _sc_router_body(scores_ref, topi_ref, topv_ref, s_v, i_v, v_v):
    wid = lax.axis_index("s") * NC + lax.axis_index("c")
    base = wid * TPW
    pltpu.sync_copy(scores_ref.at[:, pl.ds(base, TPW)], s_v)

    def group(g, _):
        t0 = g * L
        m1 = s_v[0, pl.ds(t0, L)]
        i1 = jnp.zeros((L,), jnp.int32)
        m2 = jnp.full((L,), -3.4e38, jnp.float32)
        i2 = jnp.zeros((L,), jnp.int32)

        def estep(e, c):
            m1, i1, m2, i2 = c
            v = s_v[e, pl.ds(t0, L)]
            es = jnp.full((L,), 1, jnp.int32) * e
            gt1 = v > m1
            gt2 = v > m2
            i2n = jnp.where(gt1, i1, jnp.where(gt2, es, i2))
            m2n = jnp.where(gt1, m1, jnp.where(gt2, v, m2))
            i1n = jnp.where(gt1, es, i1)
            m1n = jnp.where(gt1, v, m1)
            return m1n, i1n, m2n, i2n

        m1, i1, m2, i2 = lax.fori_loop(1, N_EXPERTS, estep, (m1, i1, m2, i2))

        def zstep(e, z):
            v = s_v[e, pl.ds(t0, L)]
            return z + jnp.exp(v - m1)

        z = lax.fori_loop(0, N_EXPERTS, zstep, jnp.zeros((L,), jnp.float32))

        p1 = 1.0 / z
        p2 = jnp.exp(m2 - m1) / z
        denom = p1 + p2 + 1e-9
        i_v[0, pl.ds(t0, L)] = i1
        i_v[1, pl.ds(t0, L)] = i2
        v_v[0, pl.ds(t0, L)] = p1 / denom
        v_v[1, pl.ds(t0, L)] = p2 / denom
        return 0

    lax.fori_loop(0, GROUPS, group, 0)
    pltpu.sync_copy(i_v, topi_ref.at[:, pl.ds(base, TPW)])
    pltpu.sync_copy(v_v, topv_ref.at[:, pl.ds(base, TPW)])


_sc_router = functools.partial(
    pl.kernel,
    out_type=[
        jax.ShapeDtypeStruct((2, TOKENS), jnp.int32),
        jax.ShapeDtypeStruct((2, TOKENS), jnp.float32),
    ],
    mesh=plsc.VectorSubcoreMesh(core_axis_name="c", subcore_axis_name="s"),
    scratch_types=[
        pltpu.VMEM((N_EXPERTS, TPW), jnp.float32),
        pltpu.VMEM((2, TPW), jnp.int32),
        pltpu.VMEM((2, TPW), jnp.float32),
    ],
)(_sc_router_body)


@jax.jit
def kernel(x, W):
    scores_t = _scores_t(x, W)
    topi_t, topv_t = _sc_router(scores_t)
    return topi_t.T, topv_t.T


# speedup vs baseline: 1.0632x; 1.0632x over previous
import functools

import jax
import jax.numpy as jnp
from jax import lax
from jax.experimental import pallas as pl
from jax.experimental.pallas import tpu as pltpu
from jax.experimental.pallas import tpu_sc as plsc

TOKENS = 16384
D_MODEL = 4096
N_EXPERTS = 64
BLK = 1024

NC, NS, L = 2, 16, 16
NW = NC * NS
TPW = TOKENS // NW
GROUPS = TPW // L


def _matmul_block(x_ref, w_ref, s_ref):
    s_ref[...] = jax.lax.dot_general(
        w_ref[...], x_ref[...], (((1,), (1,)), ((), ())),
        preferred_element_type=jnp.float32,
        precision=jax.lax.Precision.DEFAULT,
    )


def _scores_t(x, W):
    return pl.pallas_call(
        _matmul_block,
        grid=(TOKENS // BLK,),
        in_specs=[
            pl.BlockSpec((BLK, D_MODEL), lambda i: (i, 0)),
            pl.BlockSpec((N_EXPERTS, D_MODEL), lambda i: (0, 0)),
        ],
        out_specs=pl.BlockSpec((N_EXPERTS, BLK), lambda i: (0, i)),
        out_shape=jax.ShapeDtypeStruct((N_EXPERTS, TOKENS), jnp.float32),
    )(x, W)


def _sc_router_body(scores_ref, topi_ref, topv_ref, s_v, i_v, v_v):
    wid = lax.axis_index("s") * NC + lax.axis_index("c")
    base = wid * TPW
    pltpu.sync_copy(scores_ref.at[:, pl.ds(base, TPW)], s_v)

    def group(g, _):
        t0 = g * L
        m1 = s_v[0, pl.ds(t0, L)]
        i1 = jnp.zeros((L,), jnp.int32)
        m2 = jnp.full((L,), -3.4e38, jnp.float32)
        i2 = jnp.zeros((L,), jnp.int32)

        for e in range(1, N_EXPERTS):
            v = s_v[e, pl.ds(t0, L)]
            es = jnp.full((L,), e, jnp.int32)
            gt1 = v > m1
            gt2 = v > m2
            i2 = jnp.where(gt1, i1, jnp.where(gt2, es, i2))
            m2 = jnp.where(gt1, m1, jnp.where(gt2, v, m2))
            i1 = jnp.where(gt1, es, i1)
            m1 = jnp.where(gt1, v, m1)

        z = jnp.zeros((L,), jnp.float32)
        for e in range(N_EXPERTS):
            v = s_v[e, pl.ds(t0, L)]
            z = z + jnp.exp(v - m1)

        p1 = 1.0 / z
        p2 = jnp.exp(m2 - m1) / z
        denom = p1 + p2 + 1e-9
        i_v[0, pl.ds(t0, L)] = i1
        i_v[1, pl.ds(t0, L)] = i2
        v_v[0, pl.ds(t0, L)] = p1 / denom
        v_v[1, pl.ds(t0, L)] = p2 / denom
        return 0

    lax.fori_loop(0, GROUPS, group, 0)
    pltpu.sync_copy(i_v, topi_ref.at[:, pl.ds(base, TPW)])
    pltpu.sync_copy(v_v, topv_ref.at[:, pl.ds(base, TPW)])


_sc_router = functools.partial(
    pl.kernel,
    out_type=[
        jax.ShapeDtypeStruct((2, TOKENS), jnp.int32),
        jax.ShapeDtypeStruct((2, TOKENS), jnp.float32),
    ],
    mesh=plsc.VectorSubcoreMesh(core_axis_name="c", subcore_axis_name="s"),
    scratch_types=[
        pltpu.VMEM((N_EXPERTS, TPW), jnp.float32),
        pltpu.VMEM((2, TPW), jnp.int32),
        pltpu.VMEM((2, TPW), jnp.float32),
    ],
)(_sc_router_body)


@jax.jit
def kernel(x, W):
    scores_t = _scores_t(x, W)
    topi_t, topv_t = _sc_router(scores_t)
    return topi_t.T, topv_t.T
